# baseline (speedup 1.0000x reference)
import jax
import jax.numpy as jnp
from jax.experimental import pallas as pl

Q = 1024
K = 100000
D = 128
TOPK = 200

KB = 2048


def _l2n(x, axis, eps=1e-12):
    n = jnp.sqrt(jnp.sum(x * x, axis=axis, keepdims=True))
    return x / jnp.maximum(n, eps)


def _mm_body(s_ref, m_ref, o_ref):
    o_ref[:, :] = jax.lax.dot_general(
        s_ref[:, :], m_ref[:, :],
        dimension_numbers=(((1,), (1,)), ((), ())),
        preferred_element_type=jnp.float32,
    )


def _matmul(sn, mn):
    grid = (pl.cdiv(K, KB),)
    return pl.pallas_call(
        _mm_body,
        grid=grid,
        in_specs=[
            pl.BlockSpec((Q, D), lambda i: (0, 0)),
            pl.BlockSpec((KB, D), lambda i: (i, 0)),
        ],
        out_specs=pl.BlockSpec((Q, KB), lambda i: (0, i)),
        out_shape=jax.ShapeDtypeStruct((Q, K), jnp.float32),
    )(sn, mn)


def kernel(state_past, memory_past):
    sn = _l2n(state_past, axis=1)
    mn = _l2n(memory_past, axis=1)
    w = _matmul(sn, mn)
    idx = jnp.argsort(-w, axis=1)
    sel = jnp.take(memory_past, idx[:, :TOPK], axis=0)
    return idx, w, sel
